# Initial kernel scaffold; baseline (speedup 1.0000x reference)
import jax
import jax.numpy as jnp
from jax.experimental import pallas as pl


def kernel(x, W_E):
    raise NotImplementedError("write your pallas kernel here")


# speedup vs baseline: 1.8386x; 1.8386x over previous
import functools

import jax
import jax.numpy as jnp
from jax import lax
from jax.experimental import pallas as pl
from jax.experimental.pallas import tpu as pltpu
from jax.experimental.pallas import tpu_sc as plsc


@functools.lru_cache(maxsize=None)
def _build_embed(B, V, D):
    info = plsc.get_sparse_core_info()
    NC, NS = info.num_cores, info.num_subcores
    NW = NC * NS
    assert B % NW == 0
    b_per_w = B // NW
    CHUNK = 128
    assert b_per_w % (2 * CHUNK) == 0
    n_pairs = b_per_w // (2 * CHUNK)

    mesh = plsc.VectorSubcoreMesh(core_axis_name="c", subcore_axis_name="s")

    @functools.partial(
        pl.kernel,
        mesh=mesh,
        out_type=jax.ShapeDtypeStruct((B, D), jnp.float32),
        scratch_types=[
            pltpu.VMEM((b_per_w,), jnp.int32),
            pltpu.VMEM((CHUNK, D), jnp.float32),
            pltpu.VMEM((CHUNK, D), jnp.float32),
            pltpu.SemaphoreType.DMA,
            pltpu.SemaphoreType.DMA,
        ],
        compiler_params=pltpu.CompilerParams(use_tc_tiling_on_sc=False),
    )
    def embed(idx_hbm, table_hbm, out_hbm, idx_v, rows0, rows1, sem0, sem1):
        wid = lax.axis_index("s") * NC + lax.axis_index("c")
        base = wid * b_per_w
        pltpu.sync_copy(idx_hbm.at[pl.ds(base, b_per_w)], idx_v)

        bufs = (rows0, rows1)
        sems = (sem0, sem1)

        def start_gather(c, b):
            off = pl.multiple_of(c * CHUNK, CHUNK)
            pltpu.async_copy(table_hbm.at[idx_v.at[pl.ds(off, CHUNK)]],
                             bufs[b], sems[b])

        def drain_gather(b):
            pltpu.make_async_copy(table_hbm.at[pl.ds(0, CHUNK)],
                                  bufs[b], sems[b]).wait()

        def store(c, b):
            off = pl.multiple_of(base + c * CHUNK, CHUNK)
            pltpu.sync_copy(bufs[b], out_hbm.at[pl.ds(off, CHUNK)])

        start_gather(0, 0)

        def body(j, carry):
            c0 = j * 2
            for b in range(2):
                c = c0 + b
                nxt = c + 1

                @pl.when(nxt < 2 * n_pairs)
                def _():
                    start_gather(nxt, 1 - b)

                drain_gather(b)
                store(c, b)
            return carry

        lax.fori_loop(0, n_pairs, body, 0)

    return embed


def kernel(x, W_E):
    S0, S1 = x.shape
    V, D = W_E.shape
    B = S0 * S1
    idx = x.reshape(B).astype(jnp.int32)
    out = _build_embed(B, V, D)(idx, W_E)
    return out.reshape(S0, S1, D)


# speedup vs baseline: 2.0261x; 1.1019x over previous
import functools

import jax
import jax.numpy as jnp
from jax import lax
from jax.experimental import pallas as pl
from jax.experimental.pallas import tpu as pltpu
from jax.experimental.pallas import tpu_sc as plsc


@functools.lru_cache(maxsize=None)
def _build_embed(B, V, D):
    info = plsc.get_sparse_core_info()
    NC, NS = info.num_cores, info.num_subcores
    NW = NC * NS
    assert B % NW == 0
    b_per_w = B // NW
    CHUNK = 128
    NBUF = 8
    LOOKAHEAD = 6
    assert b_per_w % (NBUF * CHUNK) == 0
    n_chunks = b_per_w // CHUNK
    n_rounds = n_chunks // NBUF

    mesh = plsc.VectorSubcoreMesh(core_axis_name="c", subcore_axis_name="s")

    @functools.partial(
        pl.kernel,
        mesh=mesh,
        out_type=jax.ShapeDtypeStruct((B, D), jnp.float32),
        scratch_types=(
            [pltpu.VMEM((b_per_w,), jnp.int32)]
            + [pltpu.VMEM((CHUNK, D), jnp.float32) for _ in range(NBUF)]
            + [pltpu.SemaphoreType.DMA for _ in range(2 * NBUF)]
        ),
        compiler_params=pltpu.CompilerParams(use_tc_tiling_on_sc=False),
    )
    def embed(idx_hbm, table_hbm, out_hbm, idx_v, *bufs_sems):
        bufs = bufs_sems[:NBUF]
        gsems = bufs_sems[NBUF:2 * NBUF]
        ssems = bufs_sems[2 * NBUF:]

        wid = lax.axis_index("s") * NC + lax.axis_index("c")
        base = wid * b_per_w
        pltpu.sync_copy(idx_hbm.at[pl.ds(base, b_per_w)], idx_v)

        def start_gather(c, b):
            off = pl.multiple_of(c * CHUNK, CHUNK)
            pltpu.async_copy(table_hbm.at[idx_v.at[pl.ds(off, CHUNK)]],
                             bufs[b], gsems[b])

        def drain_gather(b):
            pltpu.make_async_copy(table_hbm.at[pl.ds(0, CHUNK)],
                                  bufs[b], gsems[b]).wait()

        def start_store(c, b):
            off = pl.multiple_of(base + c * CHUNK, CHUNK)
            pltpu.async_copy(bufs[b], out_hbm.at[pl.ds(off, CHUNK)], ssems[b])

        def drain_store(b):
            pltpu.make_async_copy(bufs[b], out_hbm.at[pl.ds(0, CHUNK)],
                                  ssems[b]).wait()

        for c in range(LOOKAHEAD):
            start_gather(c, c % NBUF)

        def body(j, carry):
            c0 = j * NBUF
            for b in range(NBUF):
                c = c0 + b
                drain_gather(b)
                start_store(c, b)
                nc = c + LOOKAHEAD
                bn = (b + LOOKAHEAD) % NBUF

                @pl.when(nc < n_chunks)
                def _():
                    @pl.when(nc >= NBUF)
                    def _():
                        drain_store(bn)

                    start_gather(nc, bn)
            return carry

        lax.fori_loop(0, n_rounds, body, 0)

        for b in range(NBUF):
            drain_store(b)

    return embed


def kernel(x, W_E):
    S0, S1 = x.shape
    V, D = W_E.shape
    B = S0 * S1
    idx = x.reshape(B).astype(jnp.int32)
    out = _build_embed(B, V, D)(idx, W_E)
    return out.reshape(S0, S1, D)
